# baseline (device time: 179482 ns/iter reference)
import jax
import jax.numpy as jnp
from jax import lax
from jax.experimental import pallas as pl
from jax.experimental.pallas import tpu as pltpu

N_DEV = 4
SQ = 1024
SKV = 1024
HQ_LOCAL = 8
DH = 128
D_MODEL = 1024
D_LOCAL = HQ_LOCAL * DH
BLK = 64
SCALE = 0.08838834764831843


def _body(x_ref, wq_ref, k_ref, v_ref, wo_ref, out_ref,
          comm_ref, send_sems, recv_sems):
    my = lax.axis_index("i")
    left = (my + N_DEV - 1) % N_DEV
    right = (my + 1) % N_DEV

    barrier = pltpu.get_barrier_semaphore()
    for nbr in (left, right):
        pl.semaphore_signal(barrier, inc=1, device_id=(nbr,),
                            device_id_type=pl.DeviceIdType.MESH)
    pl.semaphore_wait(barrier, 2)

    q = jnp.dot(x_ref[...], wq_ref[...], preferred_element_type=jnp.float32)
    q = (q * SCALE).astype(jnp.bfloat16)

    rowb = lax.broadcasted_iota(jnp.int32, (SQ, SKV), 0) // BLK
    colb = lax.broadcasted_iota(jnp.int32, (SQ, SKV), 1) // BLK
    mask = colb <= rowb

    acc = jnp.zeros((SQ, D_MODEL), jnp.float32)
    for h in range(HQ_LOCAL):
        qh = q[:, h * DH:(h + 1) * DH]
        kh = k_ref[:, h * DH:(h + 1) * DH]
        s = lax.dot_general(qh, kh, (((1,), (1,)), ((), ())),
                            preferred_element_type=jnp.float32)
        s = jnp.where(mask, s, -1e9)
        m = jnp.max(s, axis=1, keepdims=True)
        w = jnp.exp(s - m)
        w = (w / jnp.sum(w, axis=1, keepdims=True)).astype(jnp.bfloat16)
        ctx = jnp.dot(w, v_ref[:, h * DH:(h + 1) * DH],
                      preferred_element_type=jnp.float32).astype(jnp.bfloat16)
        acc = acc + jnp.dot(ctx, wo_ref[h * DH:(h + 1) * DH, :],
                            preferred_element_type=jnp.float32)

    comm_ref[0] = acc
    out = acc
    for h in range(N_DEV - 1):
        rdma = pltpu.make_async_remote_copy(
            src_ref=comm_ref.at[h],
            dst_ref=comm_ref.at[h + 1],
            send_sem=send_sems.at[h],
            recv_sem=recv_sems.at[h],
            device_id=(right,),
            device_id_type=pl.DeviceIdType.MESH,
        )
        rdma.start()
        rdma.wait()
        out = out + comm_ref[h + 1]
    out_ref[0] = out


def kernel(x, Wq, K_ext, V_ext, Wo):
    idx = lax.axis_index("i")
    x2 = x[0].astype(jnp.bfloat16)
    wq_l = lax.dynamic_slice(Wq, (0, idx * D_LOCAL),
                             (D_MODEL, D_LOCAL)).astype(jnp.bfloat16)
    wo_l = lax.dynamic_slice(Wo, (idx * D_LOCAL, 0),
                             (D_LOCAL, D_MODEL)).astype(jnp.bfloat16)
    k2 = K_ext[0].reshape(SKV, D_LOCAL).astype(jnp.bfloat16)
    v2 = V_ext[0].reshape(SKV, D_LOCAL).astype(jnp.bfloat16)

    return pl.pallas_call(
        _body,
        out_shape=jax.ShapeDtypeStruct((1, SQ, D_MODEL), jnp.float32),
        in_specs=[pl.BlockSpec(memory_space=pltpu.VMEM)] * 5,
        out_specs=pl.BlockSpec(memory_space=pltpu.VMEM),
        scratch_shapes=[
            pltpu.VMEM((N_DEV, SQ, D_MODEL), jnp.float32),
            pltpu.SemaphoreType.DMA((N_DEV - 1,)),
            pltpu.SemaphoreType.DMA((N_DEV - 1,)),
        ],
        compiler_params=pltpu.CompilerParams(collective_id=0),
    )(x2, wq_l, k2, v2, wo_l)


# device time: 82408 ns/iter; 2.1780x vs baseline; 2.1780x over previous
import jax
import jax.numpy as jnp
from jax import lax
from jax.experimental import pallas as pl
from jax.experimental.pallas import tpu as pltpu

N_DEV = 4
SQ = 1024
SKV = 1024
HQ_LOCAL = 8
DH = 128
D_MODEL = 1024
D_LOCAL = HQ_LOCAL * DH
BLK = 64
CH = SQ // N_DEV
SCALE = 0.08838834764831843


def _body(x_ref, wq_ref, k_ref, v_ref, wo_ref, out_ref,
          part_ref, send_ref, rsrecv_ref, ag_ref,
          rs_send_sems, rs_recv_sems, ag_send_sems, ag_recv_sems):
    my = lax.axis_index("i")
    left = (my + N_DEV - 1) % N_DEV
    right = (my + 1) % N_DEV

    barrier = pltpu.get_barrier_semaphore()
    for nbr in (left, right):
        pl.semaphore_signal(barrier, inc=1, device_id=(nbr,),
                            device_id_type=pl.DeviceIdType.MESH)
    pl.semaphore_wait(barrier, 2)

    q = jnp.dot(x_ref[...], wq_ref[...], preferred_element_type=jnp.float32)
    q = (q * SCALE).astype(jnp.bfloat16)

    rowb = lax.broadcasted_iota(jnp.int32, (SQ, SKV), 0) // BLK
    colb = lax.broadcasted_iota(jnp.int32, (SQ, SKV), 1) // BLK
    mask = colb <= rowb

    acc = jnp.zeros((SQ, D_MODEL), jnp.float32)
    for h in range(HQ_LOCAL):
        qh = q[:, h * DH:(h + 1) * DH]
        kh = k_ref[:, h * DH:(h + 1) * DH]
        s = lax.dot_general(qh, kh, (((1,), (1,)), ((), ())),
                            preferred_element_type=jnp.float32)
        s = jnp.where(mask, s, -1e9)
        m = jnp.max(s, axis=1, keepdims=True)
        w = jnp.exp(s - m)
        w = (w / jnp.sum(w, axis=1, keepdims=True)).astype(jnp.bfloat16)
        ctx = jnp.dot(w, v_ref[:, h * DH:(h + 1) * DH],
                      preferred_element_type=jnp.float32).astype(jnp.bfloat16)
        acc = acc + jnp.dot(ctx, wo_ref[h * DH:(h + 1) * DH, :],
                            preferred_element_type=jnp.float32)
    part_ref[...] = acc

    send_ref[0] = part_ref[pl.ds((my % N_DEV) * CH, CH), :].astype(jnp.bfloat16)
    for s in range(N_DEV - 1):
        rdma = pltpu.make_async_remote_copy(
            src_ref=send_ref.at[s],
            dst_ref=rsrecv_ref.at[s],
            send_sem=rs_send_sems.at[s],
            recv_sem=rs_recv_sems.at[s],
            device_id=(right,),
            device_id_type=pl.DeviceIdType.MESH,
        )
        rdma.start()
        rdma.wait()
        c = (my + N_DEV - s - 1) % N_DEV
        summed = (part_ref[pl.ds(c * CH, CH), :]
                  + rsrecv_ref[s].astype(jnp.float32))
        if s < N_DEV - 2:
            send_ref[s + 1] = summed.astype(jnp.bfloat16)
        else:
            ag_ref[0] = summed.astype(jnp.bfloat16)
            out_ref[0, pl.ds(c * CH, CH), :] = summed

    for t in range(N_DEV - 1):
        rdma = pltpu.make_async_remote_copy(
            src_ref=ag_ref.at[t],
            dst_ref=ag_ref.at[t + 1],
            send_sem=ag_send_sems.at[t],
            recv_sem=ag_recv_sems.at[t],
            device_id=(right,),
            device_id_type=pl.DeviceIdType.MESH,
        )
        rdma.start()
        rdma.wait()
        c = (my + N_DEV - t) % N_DEV
        out_ref[0, pl.ds(c * CH, CH), :] = ag_ref[t + 1].astype(jnp.float32)


def kernel(x, Wq, K_ext, V_ext, Wo):
    idx = lax.axis_index("i")
    x2 = x[0].astype(jnp.bfloat16)
    wq_l = lax.dynamic_slice(Wq, (0, idx * D_LOCAL),
                             (D_MODEL, D_LOCAL)).astype(jnp.bfloat16)
    wo_l = lax.dynamic_slice(Wo, (idx * D_LOCAL, 0),
                             (D_LOCAL, D_MODEL)).astype(jnp.bfloat16)
    k2 = K_ext[0].reshape(SKV, D_LOCAL).astype(jnp.bfloat16)
    v2 = V_ext[0].reshape(SKV, D_LOCAL).astype(jnp.bfloat16)

    return pl.pallas_call(
        _body,
        out_shape=jax.ShapeDtypeStruct((1, SQ, D_MODEL), jnp.float32),
        in_specs=[pl.BlockSpec(memory_space=pltpu.VMEM)] * 5,
        out_specs=pl.BlockSpec(memory_space=pltpu.VMEM),
        scratch_shapes=[
            pltpu.VMEM((SQ, D_MODEL), jnp.float32),
            pltpu.VMEM((N_DEV - 1, CH, D_MODEL), jnp.bfloat16),
            pltpu.VMEM((N_DEV - 1, CH, D_MODEL), jnp.bfloat16),
            pltpu.VMEM((N_DEV, CH, D_MODEL), jnp.bfloat16),
            pltpu.SemaphoreType.DMA((N_DEV - 1,)),
            pltpu.SemaphoreType.DMA((N_DEV - 1,)),
            pltpu.SemaphoreType.DMA((N_DEV - 1,)),
            pltpu.SemaphoreType.DMA((N_DEV - 1,)),
        ],
        compiler_params=pltpu.CompilerParams(collective_id=0),
    )(x2, wq_l, k2, v2, wo_l)


# device time: 65373 ns/iter; 2.7455x vs baseline; 1.2606x over previous
import jax
import jax.numpy as jnp
from jax import lax
from jax.experimental import pallas as pl
from jax.experimental.pallas import tpu as pltpu

N_DEV = 4
SQ = 1024
SKV = 1024
HQ_LOCAL = 8
DH = 128
D_MODEL = 1024
D_LOCAL = HQ_LOCAL * DH
BLK = 64
CH = SQ // N_DEV
SCALE = 0.08838834764831843


def _body(x_ref, wq_ref, k_ref, v_ref, wo_ref, out_ref,
          partbf_ref, rs_recv_ref, red_ref, ag_recv_ref,
          rs_send_sems, rs_recv_sems, ag_send_sems, ag_recv_sems):
    my = lax.axis_index("i")

    barrier = pltpu.get_barrier_semaphore()
    for o in (1, 2, 3):
        pl.semaphore_signal(barrier, inc=1, device_id=((my + o) % N_DEV,),
                            device_id_type=pl.DeviceIdType.MESH)
    pl.semaphore_wait(barrier, 3)

    q = jnp.dot(x_ref[...], wq_ref[...], preferred_element_type=jnp.float32)
    q = (q * SCALE).astype(jnp.bfloat16)

    rowb = lax.broadcasted_iota(jnp.int32, (SQ, SKV), 0) // BLK
    colb = lax.broadcasted_iota(jnp.int32, (SQ, SKV), 1) // BLK
    mask = colb <= rowb

    acc = jnp.zeros((SQ, D_MODEL), jnp.float32)
    for h in range(HQ_LOCAL):
        qh = q[:, h * DH:(h + 1) * DH]
        kh = k_ref[:, h * DH:(h + 1) * DH]
        s = lax.dot_general(qh, kh, (((1,), (1,)), ((), ())),
                            preferred_element_type=jnp.float32)
        s = jnp.where(mask, s, -1e9)
        m = jnp.max(s, axis=1, keepdims=True)
        w = jnp.exp(s - m)
        w = (w / jnp.sum(w, axis=1, keepdims=True)).astype(jnp.bfloat16)
        ctx = jnp.dot(w, v_ref[:, h * DH:(h + 1) * DH],
                      preferred_element_type=jnp.float32).astype(jnp.bfloat16)
        acc = acc + jnp.dot(ctx, wo_ref[h * DH:(h + 1) * DH, :],
                            preferred_element_type=jnp.float32)
    for j in range(N_DEV):
        partbf_ref[j] = acc[j * CH:(j + 1) * CH, :].astype(jnp.bfloat16)

    rs_sends = []
    for o in (1, 2, 3):
        peer = (my + o) % N_DEV
        rdma = pltpu.make_async_remote_copy(
            src_ref=partbf_ref.at[peer],
            dst_ref=rs_recv_ref.at[3 - o],
            send_sem=rs_send_sems.at[o - 1],
            recv_sem=rs_recv_sems.at[3 - o],
            device_id=(peer,),
            device_id_type=pl.DeviceIdType.MESH,
        )
        rdma.start()
        rs_sends.append(rdma)
    for rdma in rs_sends:
        rdma.wait_send()
    for slot in range(3):
        pltpu.make_async_remote_copy(
            src_ref=rs_recv_ref.at[slot],
            dst_ref=rs_recv_ref.at[slot],
            send_sem=rs_send_sems.at[0],
            recv_sem=rs_recv_sems.at[slot],
            device_id=(my,),
            device_id_type=pl.DeviceIdType.MESH,
        ).wait_recv()

    red = (partbf_ref[my].astype(jnp.float32)
           + rs_recv_ref[0].astype(jnp.float32)
           + rs_recv_ref[1].astype(jnp.float32)
           + rs_recv_ref[2].astype(jnp.float32))
    out_ref[0, pl.ds(my * CH, CH), :] = red
    red_ref[...] = red.astype(jnp.bfloat16)

    ag_sends = []
    for o in (1, 2, 3):
        peer = (my + o) % N_DEV
        rdma = pltpu.make_async_remote_copy(
            src_ref=red_ref,
            dst_ref=ag_recv_ref.at[3 - o],
            send_sem=ag_send_sems.at[o - 1],
            recv_sem=ag_recv_sems.at[3 - o],
            device_id=(peer,),
            device_id_type=pl.DeviceIdType.MESH,
        )
        rdma.start()
        ag_sends.append(rdma)
    for slot in range(3):
        pltpu.make_async_remote_copy(
            src_ref=ag_recv_ref.at[slot],
            dst_ref=ag_recv_ref.at[slot],
            send_sem=ag_send_sems.at[0],
            recv_sem=ag_recv_sems.at[slot],
            device_id=(my,),
            device_id_type=pl.DeviceIdType.MESH,
        ).wait_recv()
        c = (my + slot + 1) % N_DEV
        out_ref[0, pl.ds(c * CH, CH), :] = ag_recv_ref[slot].astype(jnp.float32)
    for rdma in ag_sends:
        rdma.wait_send()


def kernel(x, Wq, K_ext, V_ext, Wo):
    idx = lax.axis_index("i")
    x2 = x[0].astype(jnp.bfloat16)
    wq_l = lax.dynamic_slice(Wq, (0, idx * D_LOCAL),
                             (D_MODEL, D_LOCAL)).astype(jnp.bfloat16)
    wo_l = lax.dynamic_slice(Wo, (idx * D_LOCAL, 0),
                             (D_LOCAL, D_MODEL)).astype(jnp.bfloat16)
    k2 = K_ext[0].reshape(SKV, D_LOCAL).astype(jnp.bfloat16)
    v2 = V_ext[0].reshape(SKV, D_LOCAL).astype(jnp.bfloat16)

    return pl.pallas_call(
        _body,
        out_shape=jax.ShapeDtypeStruct((1, SQ, D_MODEL), jnp.float32),
        in_specs=[pl.BlockSpec(memory_space=pltpu.VMEM)] * 5,
        out_specs=pl.BlockSpec(memory_space=pltpu.VMEM),
        scratch_shapes=[
            pltpu.VMEM((N_DEV, CH, D_MODEL), jnp.bfloat16),
            pltpu.VMEM((N_DEV - 1, CH, D_MODEL), jnp.bfloat16),
            pltpu.VMEM((CH, D_MODEL), jnp.bfloat16),
            pltpu.VMEM((N_DEV - 1, CH, D_MODEL), jnp.bfloat16),
            pltpu.SemaphoreType.DMA((N_DEV - 1,)),
            pltpu.SemaphoreType.DMA((N_DEV - 1,)),
            pltpu.SemaphoreType.DMA((N_DEV - 1,)),
            pltpu.SemaphoreType.DMA((N_DEV - 1,)),
        ],
        compiler_params=pltpu.CompilerParams(collective_id=0),
    )(x2, wq_l, k2, v2, wo_l)
